# baseline (speedup 1.0000x reference)
import functools

import jax
import jax.numpy as jnp
from jax import lax
from jax.experimental import pallas as pl
from jax.experimental.pallas import tpu as pltpu
from jax.experimental.pallas import tpu_sc as plsc

BATCH = 16384
EMBED = 64
LANES = 16

_info = plsc.get_sparse_core_info()
_NC = _info.num_cores
_NS = _info.num_subcores
_NW = _NC * _NS
_BPW = BATCH // _NW
_PASS = 256
_NPASS = _BPW // _PASS
_CHUNK = 128
_NCHUNK = _BPW // _CHUNK

_mesh = plsc.VectorSubcoreMesh(core_axis_name="c", subcore_axis_name="s")


@functools.partial(
    pl.kernel,
    mesh=_mesh,
    out_type=jax.ShapeDtypeStruct((EMBED, BATCH), jnp.float32),
    compiler_params=pltpu.CompilerParams(needs_layout_passes=False),
    scratch_types=[
        pltpu.VMEM((_NCHUNK, _CHUNK), jnp.int32),
        pltpu.VMEM((_NCHUNK, _CHUNK), jnp.int32),
        pltpu.VMEM((_BPW,), jnp.int32),
        pltpu.VMEM((_BPW,), jnp.int32),
        pltpu.VMEM((_PASS, 2 * EMBED), jnp.float32),
        pltpu.VMEM((_PASS, 2 * EMBED), jnp.float32),
        pltpu.VMEM((EMBED, _BPW), jnp.float32),
        pltpu.SemaphoreType.DMA,
    ],
)
def _gmf_sc(uprow_hbm, iprow_hbm, uoff_hbm, ioff_hbm, upk_hbm, ipk_hbm, out_hbm,
            uprow_v, iprow_v, uoff_v, ioff_v, ublk, iblk, outb, sem):
    wid = lax.axis_index("s") * _NC + lax.axis_index("c")
    base = wid * _BPW

    pltpu.sync_copy(uprow_hbm.at[wid], uprow_v)
    pltpu.sync_copy(iprow_hbm.at[wid], iprow_v)
    pltpu.sync_copy(uoff_hbm.at[wid], uoff_v)
    pltpu.sync_copy(ioff_hbm.at[wid], ioff_v)

    for p in range(_NPASS):
        copies = []
        for j in range(_PASS // _CHUNK):
            c = p * (_PASS // _CHUNK) + j
            dst = ublk.at[pl.ds(j * _CHUNK, _CHUNK)]
            copies.append(pltpu.async_copy(upk_hbm.at[uprow_v.at[c]], dst, sem))
            dst = iblk.at[pl.ds(j * _CHUNK, _CHUNK)]
            copies.append(pltpu.async_copy(ipk_hbm.at[iprow_v.at[c]], dst, sem))
        for cp in copies:
            cp.wait()

        hv = lax.iota(jnp.int32, LANES)

        def g_body(g, _):
            b0 = p * _PASS + g * LANES
            uo = uoff_v[pl.ds(b0, LANES)]
            io = ioff_v[pl.ds(b0, LANES)]
            ub = ublk.at[pl.ds(g * LANES, LANES)]
            ib = iblk.at[pl.ds(g * LANES, LANES)]

            def e_body(e, _):
                evec = jnp.full((LANES,), e, jnp.int32)
                uval = plsc.load_gather(ub, [hv, uo + evec])
                ival = plsc.load_gather(ib, [hv, io + evec])
                outb[e, pl.ds(b0, LANES)] = uval * ival
                return 0

            lax.fori_loop(0, EMBED, e_body, 0)
            return 0

        lax.fori_loop(0, _PASS // LANES, g_body, 0)

    pltpu.sync_copy(outb, out_hbm.at[:, pl.ds(base, _BPW)])


def kernel(user_indices, item_indices, user_table, item_table):
    ui = user_indices.astype(jnp.int32)
    ii = item_indices.astype(jnp.int32)
    uprow = (ui >> 1).reshape(_NW, _NCHUNK, _CHUNK)
    iprow = (ii >> 1).reshape(_NW, _NCHUNK, _CHUNK)
    uoff = ((ui & 1) * EMBED).reshape(_NW, _BPW)
    ioff = ((ii & 1) * EMBED).reshape(_NW, _BPW)
    upk = user_table.reshape(user_table.shape[0] // 2, 2 * EMBED)
    ipk = item_table.reshape(item_table.shape[0] // 2, 2 * EMBED)
    out_t = _gmf_sc(uprow, iprow, uoff, ioff, upk, ipk)
    return out_t.T
